# baseline (speedup 1.0000x reference)
import jax
import jax.numpy as jnp
from jax import lax
from jax.experimental import pallas as pl
from jax.experimental.pallas import tpu as pltpu
from jax.experimental.pallas import tpu_sc as plsc

N = 10000
E = 160000
D_IN = 128
D_EDGE = 16
D = 32
D_EH = 128
DP = 128
N_MP = 6
N_S2S = 6

NC = 2
NS = 16
NW = NC * NS
CH = 128
NCHUNK = E // CH
BASE_CH = NCHUNK // NW
REM = NCHUNK - NW * BASE_CH
OCT = N // 8
BASE_OCT = OCT // NS
REM_OCT = OCT - NS * BASE_OCT
ROWS_SUB = BASE_OCT * 8

EB = 640
NB = 2000

_SC_MESH = plsc.VectorSubcoreMesh(core_axis_name="c", subcore_axis_name="s")


def _gather_body(x_hbm, src_hbm, out_hbm, idx_v, rows_v, sem):
    c = lax.axis_index("c")
    s = lax.axis_index("s")
    w = c * NS + s
    n_ch = BASE_CH + jnp.where(w < REM, 1, 0)
    base = w * BASE_CH + jnp.minimum(w, REM)

    def body(i, carry):
        off = (base + i) * CH
        pltpu.sync_copy(src_hbm.at[pl.ds(off, CH)], idx_v)
        pltpu.async_copy(x_hbm.at[idx_v], rows_v, sem).wait()
        pltpu.sync_copy(rows_v, out_hbm.at[pl.ds(off, CH)])
        return carry

    lax.fori_loop(0, n_ch, body, 0)


_sc_gather_call = pl.kernel(
    _gather_body,
    out_type=jax.ShapeDtypeStruct((E, D), jnp.float32),
    mesh=_SC_MESH,
    compiler_params=pltpu.CompilerParams(use_tc_tiling_on_sc=False),
    scratch_types=[
        pltpu.VMEM((CH,), jnp.int32),
        pltpu.VMEM((CH, D), jnp.float32),
        pltpu.SemaphoreType.DMA,
    ],
)


def _sc_gather(x, src):
    return _sc_gather_call(x, src)


def _scatter_body(m_hbm, dst_hbm, zeros_hbm, out_hbm, idx_v, rows_v, agg_sh):
    c = lax.axis_index("c")
    s = lax.axis_index("s")
    w = c * NS + s
    row0 = (s * BASE_OCT + jnp.minimum(s, REM_OCT)) * 8
    pltpu.sync_copy(zeros_hbm.at[pl.ds(row0, ROWS_SUB)],
                    agg_sh.at[pl.ds(row0, ROWS_SUB)])

    @pl.when(s < REM_OCT)
    def _zero_extra():
        pltpu.sync_copy(zeros_hbm.at[pl.ds(row0 + ROWS_SUB, 8)],
                        agg_sh.at[pl.ds(row0 + ROWS_SUB, 8)])

    plsc.subcore_barrier()

    n_ch = BASE_CH + jnp.where(w < REM, 1, 0)
    base = w * BASE_CH + jnp.minimum(w, REM)

    def body(i, carry):
        off = (base + i) * CH
        pltpu.sync_copy(dst_hbm.at[pl.ds(off, CH)], idx_v)
        pltpu.sync_copy(m_hbm.at[pl.ds(off, CH)], rows_v)
        pltpu.sync_copy(rows_v, agg_sh.at[idx_v], add=True)
        return carry

    lax.fori_loop(0, n_ch, body, 0)
    plsc.subcore_barrier()
    pltpu.sync_copy(agg_sh.at[pl.ds(row0, ROWS_SUB)],
                    out_hbm.at[c, pl.ds(row0, ROWS_SUB)])

    @pl.when(s < REM_OCT)
    def _write_extra():
        pltpu.sync_copy(agg_sh.at[pl.ds(row0 + ROWS_SUB, 8)],
                        out_hbm.at[c, pl.ds(row0 + ROWS_SUB, 8)])


_sc_scatter_call = pl.kernel(
    _scatter_body,
    out_type=jax.ShapeDtypeStruct((NC, N, D), jnp.float32),
    mesh=_SC_MESH,
    compiler_params=pltpu.CompilerParams(use_tc_tiling_on_sc=False),
    scratch_types=[
        pltpu.VMEM((CH,), jnp.int32),
        pltpu.VMEM((CH, D), jnp.float32),
        pltpu.VMEM_SHARED((N, D), jnp.float32),
    ],
)


def _sc_scatter(m, dst, zeros):
    return _sc_scatter_call(m, dst, zeros)


def _proj_body(nf_ref, wpt_ref, bp_ref, out_ref):
    out_ref[...] = jnp.maximum(nf_ref[...] @ wpt_ref[...] + bp_ref[...], 0.0)


def _tc_proj(node_feats, WpT, bp_row):
    return pl.pallas_call(
        _proj_body,
        grid=(N // NB,),
        in_specs=[
            pl.BlockSpec((NB, D_IN), lambda i: (i, 0)),
            pl.BlockSpec((D_IN, D), lambda i: (0, 0)),
            pl.BlockSpec((1, D), lambda i: (0, 0)),
        ],
        out_specs=pl.BlockSpec((NB, D), lambda i: (i, 0)),
        out_shape=jax.ShapeDtypeStruct((N, D), jnp.float32),
    )(node_feats, WpT, bp_row)


def _msg_body(ef_ref, xs_ref, w1_ref, b1_ref, w2_ref, t_ref, s_ref,
              brow_ref, out_ref):
    dg = lambda a, b: lax.dot_general(a, b, (((1,), (0,)), ((), ())),
                                      preferred_element_type=jnp.float32)
    h = jnp.maximum(ef_ref[...] @ w1_ref[...] + b1_ref[...], 0.0)
    h_bf = h.astype(jnp.bfloat16)
    hh = jnp.concatenate([h_bf, h_bf], axis=1)
    p = dg(hh, w2_ref[...]) + brow_ref[...]
    xs = xs_ref[...]
    xst = dg(xs.astype(jnp.bfloat16), t_ref[...])
    q = xst * p
    out_ref[...] = dg(q, s_ref[...])


def _tc_msg(edge_feats, xs, W1T, be1_row, W2cat, Tt, Sr, Brow):
    return pl.pallas_call(
        _msg_body,
        grid=(E // EB,),
        in_specs=[
            pl.BlockSpec((EB, D_EDGE), lambda i: (i, 0)),
            pl.BlockSpec((EB, D), lambda i: (i, 0)),
            pl.BlockSpec((D_EDGE, D_EH), lambda i: (0, 0)),
            pl.BlockSpec((1, D_EH), lambda i: (0, 0)),
            pl.BlockSpec((2 * D_EH, D * D), lambda i: (0, 0)),
            pl.BlockSpec((D, D * D), lambda i: (0, 0)),
            pl.BlockSpec((D * D, D), lambda i: (0, 0)),
            pl.BlockSpec((1, D * D), lambda i: (0, 0)),
        ],
        out_specs=pl.BlockSpec((EB, D), lambda i: (i, 0)),
        out_shape=jax.ShapeDtypeStruct((E, D), jnp.float32),
    )(edge_feats, xs, W1T, be1_row, W2cat, Tt, Sr, Brow)


def _gru_body(agg_ref, hid_ref, bconv_ref, wih_ref, whh_ref, bih_ref,
              bhh_ref, out_ref):
    hid = hid_ref[...]
    a = jnp.maximum(agg_ref[0] + agg_ref[1] + bconv_ref[...], 0.0)
    gi = a @ wih_ref[...] + bih_ref[...]
    gh = hid @ whh_ref[...] + bhh_ref[...]
    r = jax.nn.sigmoid(gi[:, :D] + gh[:, :D])
    z = jax.nn.sigmoid(gi[:, D:2 * D] + gh[:, D:2 * D])
    n_ = jnp.tanh(gi[:, 2 * D:] + r * gh[:, 2 * D:])
    out_ref[...] = (1.0 - z) * n_ + z * hid


def _tc_gru(agg2, hid, bconv_row, WihT, WhhT, bih_row, bhh_row):
    return pl.pallas_call(
        _gru_body,
        grid=(N // NB,),
        in_specs=[
            pl.BlockSpec((NC, NB, D), lambda i: (0, i, 0)),
            pl.BlockSpec((NB, D), lambda i: (i, 0)),
            pl.BlockSpec((1, D), lambda i: (0, 0)),
            pl.BlockSpec((D, 3 * D), lambda i: (0, 0)),
            pl.BlockSpec((D, 3 * D), lambda i: (0, 0)),
            pl.BlockSpec((1, 3 * D), lambda i: (0, 0)),
            pl.BlockSpec((1, 3 * D), lambda i: (0, 0)),
        ],
        out_specs=pl.BlockSpec((NB, D), lambda i: (i, 0)),
        out_shape=jax.ShapeDtypeStruct((N, D), jnp.float32),
    )(agg2, hid, bconv_row, WihT, WhhT, bih_row, bhh_row)


def _s2s_body(no_ref, w_ih0, w_hh0, b_ih0, b_hh0, w_ih1, w_hh1, b_ih1, b_hh1,
              w_ih2, w_hh2, b_ih2, b_hh2, wo1_ref, bo1_ref, wo2_ref, bo2_ref,
              out_ref):
    no = no_ref[...]
    wih = [w_ih0[...], w_ih1[...], w_ih2[...]]
    whh = [w_hh0[...], w_hh1[...], w_hh2[...]]
    bih = [b_ih0[...], b_ih1[...], b_ih2[...]]
    bhh = [b_hh0[...], b_hh1[...], b_hh2[...]]
    hs = [jnp.zeros((1, D), jnp.float32) for _ in range(3)]
    cs = [jnp.zeros((1, D), jnp.float32) for _ in range(3)]
    q_star = jnp.zeros((1, 2 * D), jnp.float32)
    for _ in range(N_S2S):
        xq = q_star
        for l in range(3):
            gates = xq @ wih[l] + bih[l] + hs[l] @ whh[l] + bhh[l]
            ii = jax.nn.sigmoid(gates[:, :D])
            ff = jax.nn.sigmoid(gates[:, D:2 * D])
            gg = jnp.tanh(gates[:, 2 * D:3 * D])
            oo = jax.nn.sigmoid(gates[:, 3 * D:])
            cs[l] = ff * cs[l] + ii * gg
            hs[l] = oo * jnp.tanh(cs[l])
            xq = hs[l]
        q = xq
        e = lax.dot_general(no, q, (((1,), (1,)), ((), ())))
        al = jnp.exp(e - jnp.max(e))
        al = al / jnp.sum(al)
        readout = lax.dot_general(al, no, (((0,), (0,)), ((), ())))
        q_star = jnp.concatenate([q, readout], axis=1)
    head = jnp.maximum(q_star @ wo1_ref[...] + bo1_ref[...], 0.0)
    out_ref[...] = head @ wo2_ref[...] + bo2_ref[...]


def _tc_s2s(node_out, lstm_t, Wo1T, bo1_row, Wo2T, bo2_row):
    args = [node_out]
    in_specs = [pl.BlockSpec((N, D), lambda: (0, 0))]
    for (wih, whh, bih, bhh) in lstm_t:
        args += [wih, whh, bih, bhh]
        in_specs += [pl.BlockSpec(wih.shape, lambda: (0, 0)),
                     pl.BlockSpec(whh.shape, lambda: (0, 0)),
                     pl.BlockSpec(bih.shape, lambda: (0, 0)),
                     pl.BlockSpec(bhh.shape, lambda: (0, 0))]
    args += [Wo1T, bo1_row, Wo2T, bo2_row]
    in_specs += [pl.BlockSpec(Wo1T.shape, lambda: (0, 0)),
                 pl.BlockSpec(bo1_row.shape, lambda: (0, 0)),
                 pl.BlockSpec(Wo2T.shape, lambda: (0, 0)),
                 pl.BlockSpec(bo2_row.shape, lambda: (0, 0))]
    return pl.pallas_call(
        _s2s_body,
        grid=(),
        in_specs=in_specs,
        out_specs=pl.BlockSpec((1, 2), lambda: (0, 0)),
        out_shape=jax.ShapeDtypeStruct((1, 2), jnp.float32),
    )(*args)


def kernel(node_feats, edge_feats, edge_index, Wp, bp, We1, be1, We2, be2,
           b_conv, gru_Wih, gru_Whh, gru_bih, gru_bhh,
           lstm_Wih0, lstm_Whh0, lstm_bih0, lstm_bhh0,
           lstm_Wih1, lstm_Whh1, lstm_bih1, lstm_bhh1,
           lstm_Wih2, lstm_Whh2, lstm_bih2, lstm_bhh2,
           Wo1, bo1, Wo2, bo2):
    src = edge_index[0]
    dst = edge_index[1]
    zeros = jnp.zeros((N, D), jnp.float32)

    WpT = Wp.T
    W1T = We1.T
    W2m = We2.reshape(D, D, D_EH).transpose(2, 1, 0).reshape(D_EH, D * D)
    W2_hi = W2m.astype(jnp.bfloat16)
    W2_lo = (W2m - W2_hi.astype(jnp.float32)).astype(jnp.bfloat16)
    W2cat = jnp.concatenate([W2_hi, W2_lo], axis=0)
    Tt = jnp.tile(jnp.eye(D, dtype=jnp.bfloat16), (1, D))
    Sr = jnp.repeat(jnp.eye(D, dtype=jnp.float32), D, axis=0)
    Brow = be2.reshape(D, D).T.reshape(1, D * D)
    WihT = gru_Wih.T
    WhhT = gru_Whh.T

    x = _tc_proj(node_feats, WpT, bp.reshape(1, D))
    for _ in range(N_MP):
        xs = _sc_gather(x, src)
        m = _tc_msg(edge_feats, xs, W1T, be1.reshape(1, D_EH), W2cat,
                    Tt, Sr, Brow)
        agg2 = _sc_scatter(m, dst, zeros)
        x = _tc_gru(agg2, x, b_conv.reshape(1, D), WihT, WhhT,
                    gru_bih.reshape(1, 3 * D), gru_bhh.reshape(1, 3 * D))

    lstm_t = [(lstm_Wih0.T, lstm_Whh0.T, lstm_bih0.reshape(1, 4 * D),
               lstm_bhh0.reshape(1, 4 * D)),
              (lstm_Wih1.T, lstm_Whh1.T, lstm_bih1.reshape(1, 4 * D),
               lstm_bhh1.reshape(1, 4 * D)),
              (lstm_Wih2.T, lstm_Whh2.T, lstm_bih2.reshape(1, 4 * D),
               lstm_bhh2.reshape(1, 4 * D))]
    return _tc_s2s(x, lstm_t, Wo1.T, bo1.reshape(1, D), Wo2.T,
                   bo2.reshape(1, 2))


# speedup vs baseline: 1.2120x; 1.2120x over previous
import jax
import jax.numpy as jnp
from jax import lax
from jax.experimental import pallas as pl
from jax.experimental.pallas import tpu as pltpu
from jax.experimental.pallas import tpu_sc as plsc

N = 10000
E = 160000
D_IN = 128
D_EDGE = 16
D = 32
D_EH = 128
DP = 128
N_MP = 6
N_S2S = 6

NC = 2
NS = 16
NW = NC * NS
EH = E // 2
CH = 128
NCHUNK = EH // CH
BASE_CH = NCHUNK // NW
REM = NCHUNK - NW * BASE_CH
OCT = N // 8
BASE_OCT = OCT // NS
REM_OCT = OCT - NS * BASE_OCT
ROWS_SUB = BASE_OCT * 8

EB = 640
NB = 2000

_SC_MESH = plsc.VectorSubcoreMesh(core_axis_name="c", subcore_axis_name="s")


def _gather_body(x_hbm, src_hbm, out_hbm, idx_v, rows_v, sem):
    c = lax.axis_index("c")
    s = lax.axis_index("s")
    w = c * NS + s
    n_ch = BASE_CH + jnp.where(w < REM, 1, 0)
    base = w * BASE_CH + jnp.minimum(w, REM)

    def body(i, carry):
        off = (base + i) * CH
        pltpu.sync_copy(src_hbm.at[pl.ds(off, CH)], idx_v)
        pltpu.async_copy(x_hbm.at[idx_v], rows_v, sem).wait()
        pltpu.sync_copy(rows_v, out_hbm.at[pl.ds(off, CH)])
        return carry

    lax.fori_loop(0, n_ch, body, 0)


_sc_gather_call = pl.kernel(
    _gather_body,
    out_type=jax.ShapeDtypeStruct((EH, DP), jnp.float32),
    mesh=_SC_MESH,
    scratch_types=[
        pltpu.VMEM((CH,), jnp.int32),
        pltpu.VMEM((CH, DP), jnp.float32),
        pltpu.SemaphoreType.DMA,
    ],
)


def _sc_gather(x, src):
    return _sc_gather_call(x, src)


def _scatter_body(m_hbm, dst_hbm, zeros_hbm, out_hbm, idx_v, rows_v, agg_sh):
    c = lax.axis_index("c")
    s = lax.axis_index("s")
    w = c * NS + s
    row0 = (s * BASE_OCT + jnp.minimum(s, REM_OCT)) * 8
    pltpu.sync_copy(zeros_hbm.at[pl.ds(row0, ROWS_SUB)],
                    agg_sh.at[pl.ds(row0, ROWS_SUB)])

    @pl.when(s < REM_OCT)
    def _zero_extra():
        pltpu.sync_copy(zeros_hbm.at[pl.ds(row0 + ROWS_SUB, 8)],
                        agg_sh.at[pl.ds(row0 + ROWS_SUB, 8)])

    plsc.subcore_barrier()

    n_ch = BASE_CH + jnp.where(w < REM, 1, 0)
    base = w * BASE_CH + jnp.minimum(w, REM)

    def body(i, carry):
        off = (base + i) * CH
        pltpu.sync_copy(dst_hbm.at[pl.ds(off, CH)], idx_v)
        pltpu.sync_copy(m_hbm.at[pl.ds(off, CH)], rows_v)
        pltpu.sync_copy(rows_v, agg_sh.at[idx_v], add=True)
        return carry

    lax.fori_loop(0, n_ch, body, 0)
    plsc.subcore_barrier()
    pltpu.sync_copy(agg_sh.at[pl.ds(row0, ROWS_SUB)],
                    out_hbm.at[c, pl.ds(row0, ROWS_SUB)])

    @pl.when(s < REM_OCT)
    def _write_extra():
        pltpu.sync_copy(agg_sh.at[pl.ds(row0 + ROWS_SUB, 8)],
                        out_hbm.at[c, pl.ds(row0 + ROWS_SUB, 8)])


_sc_scatter_call = pl.kernel(
    _scatter_body,
    out_type=jax.ShapeDtypeStruct((NC, N, DP), jnp.float32),
    mesh=_SC_MESH,
    scratch_types=[
        pltpu.VMEM((CH,), jnp.int32),
        pltpu.VMEM((CH, DP), jnp.float32),
        pltpu.VMEM_SHARED((N, DP), jnp.float32),
    ],
)


def _sc_scatter(m, dst, zeros):
    return _sc_scatter_call(m, dst, zeros)


def _proj_body(nf_ref, wpt_ref, bp_ref, out_ref):
    x = jnp.maximum(nf_ref[...] @ wpt_ref[...] + bp_ref[...], 0.0)
    out_ref[...] = jnp.concatenate(
        [x, jnp.zeros((x.shape[0], DP - D), jnp.float32)], axis=1)


def _tc_proj(node_feats, WpT, bp_row):
    return pl.pallas_call(
        _proj_body,
        grid=(N // NB,),
        in_specs=[
            pl.BlockSpec((NB, D_IN), lambda i: (i, 0)),
            pl.BlockSpec((D_IN, D), lambda i: (0, 0)),
            pl.BlockSpec((1, D), lambda i: (0, 0)),
        ],
        out_specs=pl.BlockSpec((NB, DP), lambda i: (i, 0)),
        out_shape=jax.ShapeDtypeStruct((N, DP), jnp.float32),
    )(node_feats, WpT, bp_row)


def _msg_body(ef_ref, xs_ref, w1_ref, b1_ref, w2_ref, t_ref, s_ref,
              brow_ref, out_ref):
    dg = lambda a, b: lax.dot_general(a, b, (((1,), (0,)), ((), ())),
                                      preferred_element_type=jnp.float32)
    h = jnp.maximum(ef_ref[...] @ w1_ref[...] + b1_ref[...], 0.0)
    h_bf = h.astype(jnp.bfloat16)
    hh = jnp.concatenate([h_bf, h_bf], axis=1)
    p = dg(hh, w2_ref[...]) + brow_ref[...]
    xs = xs_ref[:, :D]
    xst = dg(xs.astype(jnp.bfloat16), t_ref[...])
    q = xst * p
    acc = dg(q, s_ref[...])
    out_ref[...] = jnp.concatenate(
        [acc, jnp.zeros((acc.shape[0], DP - D), jnp.float32)], axis=1)


def _tc_msg(edge_feats, xs, W1T, be1_row, W2cat, Tt, Sr, Brow):
    return pl.pallas_call(
        _msg_body,
        grid=(EH // EB,),
        in_specs=[
            pl.BlockSpec((EB, D_EDGE), lambda i: (i, 0)),
            pl.BlockSpec((EB, DP), lambda i: (i, 0)),
            pl.BlockSpec((D_EDGE, D_EH), lambda i: (0, 0)),
            pl.BlockSpec((1, D_EH), lambda i: (0, 0)),
            pl.BlockSpec((2 * D_EH, D * D), lambda i: (0, 0)),
            pl.BlockSpec((D, D * D), lambda i: (0, 0)),
            pl.BlockSpec((D * D, D), lambda i: (0, 0)),
            pl.BlockSpec((1, D * D), lambda i: (0, 0)),
        ],
        out_specs=pl.BlockSpec((EB, DP), lambda i: (i, 0)),
        out_shape=jax.ShapeDtypeStruct((EH, DP), jnp.float32),
    )(edge_feats, xs, W1T, be1_row, W2cat, Tt, Sr, Brow)


def _gru_body(agga_ref, aggb_ref, hid_ref, bconv_ref, wih_ref, whh_ref,
              bih_ref, bhh_ref, out_ref):
    hid = hid_ref[:, :D]
    a = jnp.maximum(agga_ref[0, :, :D] + agga_ref[1, :, :D]
                    + aggb_ref[0, :, :D] + aggb_ref[1, :, :D]
                    + bconv_ref[...], 0.0)
    gi = a @ wih_ref[...] + bih_ref[...]
    gh = hid @ whh_ref[...] + bhh_ref[...]
    r = jax.nn.sigmoid(gi[:, :D] + gh[:, :D])
    z = jax.nn.sigmoid(gi[:, D:2 * D] + gh[:, D:2 * D])
    n_ = jnp.tanh(gi[:, 2 * D:] + r * gh[:, 2 * D:])
    x = (1.0 - z) * n_ + z * hid
    out_ref[...] = jnp.concatenate(
        [x, jnp.zeros((x.shape[0], DP - D), jnp.float32)], axis=1)


def _tc_gru(aggA, aggB, hid, bconv_row, WihT, WhhT, bih_row, bhh_row):
    return pl.pallas_call(
        _gru_body,
        grid=(N // NB,),
        in_specs=[
            pl.BlockSpec((NC, NB, DP), lambda i: (0, i, 0)),
            pl.BlockSpec((NC, NB, DP), lambda i: (0, i, 0)),
            pl.BlockSpec((NB, DP), lambda i: (i, 0)),
            pl.BlockSpec((1, D), lambda i: (0, 0)),
            pl.BlockSpec((D, 3 * D), lambda i: (0, 0)),
            pl.BlockSpec((D, 3 * D), lambda i: (0, 0)),
            pl.BlockSpec((1, 3 * D), lambda i: (0, 0)),
            pl.BlockSpec((1, 3 * D), lambda i: (0, 0)),
        ],
        out_specs=pl.BlockSpec((NB, DP), lambda i: (i, 0)),
        out_shape=jax.ShapeDtypeStruct((N, DP), jnp.float32),
    )(aggA, aggB, hid, bconv_row, WihT, WhhT, bih_row, bhh_row)


def _s2s_body(no_ref, w_ih0, w_hh0, b_ih0, b_hh0, w_ih1, w_hh1, b_ih1, b_hh1,
              w_ih2, w_hh2, b_ih2, b_hh2, wo1_ref, bo1_ref, wo2_ref, bo2_ref,
              out_ref):
    no = no_ref[:, :D]
    wih = [w_ih0[...], w_ih1[...], w_ih2[...]]
    whh = [w_hh0[...], w_hh1[...], w_hh2[...]]
    bih = [b_ih0[...], b_ih1[...], b_ih2[...]]
    bhh = [b_hh0[...], b_hh1[...], b_hh2[...]]
    hs = [jnp.zeros((1, D), jnp.float32) for _ in range(3)]
    cs = [jnp.zeros((1, D), jnp.float32) for _ in range(3)]
    q_star = jnp.zeros((1, 2 * D), jnp.float32)
    for _ in range(N_S2S):
        xq = q_star
        for l in range(3):
            gates = xq @ wih[l] + bih[l] + hs[l] @ whh[l] + bhh[l]
            ii = jax.nn.sigmoid(gates[:, :D])
            ff = jax.nn.sigmoid(gates[:, D:2 * D])
            gg = jnp.tanh(gates[:, 2 * D:3 * D])
            oo = jax.nn.sigmoid(gates[:, 3 * D:])
            cs[l] = ff * cs[l] + ii * gg
            hs[l] = oo * jnp.tanh(cs[l])
            xq = hs[l]
        q = xq
        e = lax.dot_general(no, q, (((1,), (1,)), ((), ())))
        al = jnp.exp(e - jnp.max(e))
        al = al / jnp.sum(al)
        readout = lax.dot_general(al, no, (((0,), (0,)), ((), ())))
        q_star = jnp.concatenate([q, readout], axis=1)
    head = jnp.maximum(q_star @ wo1_ref[...] + bo1_ref[...], 0.0)
    out_ref[...] = head @ wo2_ref[...] + bo2_ref[...]


def _tc_s2s(node_out, lstm_t, Wo1T, bo1_row, Wo2T, bo2_row):
    args = [node_out]
    in_specs = [pl.BlockSpec((N, DP), lambda: (0, 0))]
    for (wih, whh, bih, bhh) in lstm_t:
        args += [wih, whh, bih, bhh]
        in_specs += [pl.BlockSpec(wih.shape, lambda: (0, 0)),
                     pl.BlockSpec(whh.shape, lambda: (0, 0)),
                     pl.BlockSpec(bih.shape, lambda: (0, 0)),
                     pl.BlockSpec(bhh.shape, lambda: (0, 0))]
    args += [Wo1T, bo1_row, Wo2T, bo2_row]
    in_specs += [pl.BlockSpec(Wo1T.shape, lambda: (0, 0)),
                 pl.BlockSpec(bo1_row.shape, lambda: (0, 0)),
                 pl.BlockSpec(Wo2T.shape, lambda: (0, 0)),
                 pl.BlockSpec(bo2_row.shape, lambda: (0, 0))]
    return pl.pallas_call(
        _s2s_body,
        grid=(),
        in_specs=in_specs,
        out_specs=pl.BlockSpec((1, 2), lambda: (0, 0)),
        out_shape=jax.ShapeDtypeStruct((1, 2), jnp.float32),
    )(*args)


def kernel(node_feats, edge_feats, edge_index, Wp, bp, We1, be1, We2, be2,
           b_conv, gru_Wih, gru_Whh, gru_bih, gru_bhh,
           lstm_Wih0, lstm_Whh0, lstm_bih0, lstm_bhh0,
           lstm_Wih1, lstm_Whh1, lstm_bih1, lstm_bhh1,
           lstm_Wih2, lstm_Whh2, lstm_bih2, lstm_bhh2,
           Wo1, bo1, Wo2, bo2):
    src = edge_index[0]
    dst = edge_index[1]
    srcA, srcB = src[:EH], src[EH:]
    dstA, dstB = dst[:EH], dst[EH:]
    efA, efB = edge_feats[:EH], edge_feats[EH:]
    zeros = jnp.zeros((N, DP), jnp.float32)

    WpT = Wp.T
    W1T = We1.T
    W2m = We2.reshape(D, D, D_EH).transpose(2, 1, 0).reshape(D_EH, D * D)
    W2_hi = W2m.astype(jnp.bfloat16)
    W2_lo = (W2m - W2_hi.astype(jnp.float32)).astype(jnp.bfloat16)
    W2cat = jnp.concatenate([W2_hi, W2_lo], axis=0)
    Tt = jnp.tile(jnp.eye(D, dtype=jnp.bfloat16), (1, D))
    Sr = jnp.repeat(jnp.eye(D, dtype=jnp.float32), D, axis=0)
    Brow = be2.reshape(D, D).T.reshape(1, D * D)
    WihT = gru_Wih.T
    WhhT = gru_Whh.T

    x = _tc_proj(node_feats, WpT, bp.reshape(1, D))
    be1_row = be1.reshape(1, D_EH)
    for _ in range(N_MP):
        xsA = _sc_gather(x, srcA)
        xsB = _sc_gather(x, srcB)
        mA = _tc_msg(efA, xsA, W1T, be1_row, W2cat, Tt, Sr, Brow)
        aggA = _sc_scatter(mA, dstA, zeros)
        mB = _tc_msg(efB, xsB, W1T, be1_row, W2cat, Tt, Sr, Brow)
        aggB = _sc_scatter(mB, dstB, zeros)
        x = _tc_gru(aggA, aggB, x, b_conv.reshape(1, D), WihT, WhhT,
                    gru_bih.reshape(1, 3 * D), gru_bhh.reshape(1, 3 * D))

    lstm_t = [(lstm_Wih0.T, lstm_Whh0.T, lstm_bih0.reshape(1, 4 * D),
               lstm_bhh0.reshape(1, 4 * D)),
              (lstm_Wih1.T, lstm_Whh1.T, lstm_bih1.reshape(1, 4 * D),
               lstm_bhh1.reshape(1, 4 * D)),
              (lstm_Wih2.T, lstm_Whh2.T, lstm_bih2.reshape(1, 4 * D),
               lstm_bhh2.reshape(1, 4 * D))]
    return _tc_s2s(x, lstm_t, Wo1.T, bo1.reshape(1, D), Wo2.T,
                   bo2.reshape(1, 2))


# speedup vs baseline: 1.3076x; 1.0789x over previous
import jax
import jax.numpy as jnp
from jax import lax
from jax.experimental import pallas as pl
from jax.experimental.pallas import tpu as pltpu
from jax.experimental.pallas import tpu_sc as plsc

N = 10000
E = 160000
D_IN = 128
D_EDGE = 16
D = 32
D_EH = 128
DP = 128
N_MP = 6
N_S2S = 6

NC = 2
NS = 16
NW = NC * NS
NPART = 5
EH = E // NPART
CH = 128
NCHUNK = EH // CH
BASE_CH = NCHUNK // NW
REM = NCHUNK - NW * BASE_CH
OCT = N // 8
BASE_OCT = OCT // NS
REM_OCT = OCT - NS * BASE_OCT
ROWS_SUB = BASE_OCT * 8

EB = 640
NB = 2000

_SC_MESH = plsc.VectorSubcoreMesh(core_axis_name="c", subcore_axis_name="s")


def _gather_body(x_hbm, src_hbm, out_hbm, idx_v, rows_v, sem):
    c = lax.axis_index("c")
    s = lax.axis_index("s")
    w = c * NS + s
    n_ch = BASE_CH + jnp.where(w < REM, 1, 0)
    base = w * BASE_CH + jnp.minimum(w, REM)

    def body(i, carry):
        off = (base + i) * CH
        pltpu.sync_copy(src_hbm.at[pl.ds(off, CH)], idx_v)
        pltpu.async_copy(x_hbm.at[idx_v], rows_v, sem).wait()
        pltpu.sync_copy(rows_v, out_hbm.at[pl.ds(off, CH)])
        return carry

    lax.fori_loop(0, n_ch, body, 0)


_sc_gather_call = pl.kernel(
    _gather_body,
    out_type=jax.ShapeDtypeStruct((EH, DP), jnp.float32),
    mesh=_SC_MESH,
    scratch_types=[
        pltpu.VMEM((CH,), jnp.int32),
        pltpu.VMEM((CH, DP), jnp.float32),
        pltpu.SemaphoreType.DMA,
    ],
)


def _sc_gather(x, src):
    return _sc_gather_call(x, src)


def _scatter_body(m_hbm, dst_hbm, zeros_hbm, out_hbm, idx_v, rows_v, agg_sh):
    c = lax.axis_index("c")
    s = lax.axis_index("s")
    w = c * NS + s
    row0 = (s * BASE_OCT + jnp.minimum(s, REM_OCT)) * 8
    pltpu.sync_copy(zeros_hbm.at[pl.ds(row0, ROWS_SUB)],
                    agg_sh.at[pl.ds(row0, ROWS_SUB)])

    @pl.when(s < REM_OCT)
    def _zero_extra():
        pltpu.sync_copy(zeros_hbm.at[pl.ds(row0 + ROWS_SUB, 8)],
                        agg_sh.at[pl.ds(row0 + ROWS_SUB, 8)])

    plsc.subcore_barrier()

    n_ch = BASE_CH + jnp.where(w < REM, 1, 0)
    base = w * BASE_CH + jnp.minimum(w, REM)

    def body(i, carry):
        off = (base + i) * CH
        pltpu.sync_copy(dst_hbm.at[pl.ds(off, CH)], idx_v)
        pltpu.sync_copy(m_hbm.at[pl.ds(off, CH)], rows_v)
        pltpu.sync_copy(rows_v, agg_sh.at[idx_v], add=True)
        return carry

    lax.fori_loop(0, n_ch, body, 0)
    plsc.subcore_barrier()
    pltpu.sync_copy(agg_sh.at[pl.ds(row0, ROWS_SUB)],
                    out_hbm.at[c, pl.ds(row0, ROWS_SUB)])

    @pl.when(s < REM_OCT)
    def _write_extra():
        pltpu.sync_copy(agg_sh.at[pl.ds(row0 + ROWS_SUB, 8)],
                        out_hbm.at[c, pl.ds(row0 + ROWS_SUB, 8)])


_sc_scatter_call = pl.kernel(
    _scatter_body,
    out_type=jax.ShapeDtypeStruct((NC, N, DP), jnp.float32),
    mesh=_SC_MESH,
    scratch_types=[
        pltpu.VMEM((CH,), jnp.int32),
        pltpu.VMEM((CH, DP), jnp.float32),
        pltpu.VMEM_SHARED((N, DP), jnp.float32),
    ],
)


def _sc_scatter(m, dst, zeros):
    return _sc_scatter_call(m, dst, zeros)


def _proj_body(nf_ref, wpt_ref, bp_ref, out_ref):
    x = jnp.maximum(nf_ref[...] @ wpt_ref[...] + bp_ref[...], 0.0)
    out_ref[...] = jnp.concatenate(
        [x, jnp.zeros((x.shape[0], DP - D), jnp.float32)], axis=1)


def _tc_proj(node_feats, WpT, bp_row):
    return pl.pallas_call(
        _proj_body,
        grid=(N // NB,),
        in_specs=[
            pl.BlockSpec((NB, D_IN), lambda i: (i, 0)),
            pl.BlockSpec((D_IN, D), lambda i: (0, 0)),
            pl.BlockSpec((1, D), lambda i: (0, 0)),
        ],
        out_specs=pl.BlockSpec((NB, DP), lambda i: (i, 0)),
        out_shape=jax.ShapeDtypeStruct((N, DP), jnp.float32),
    )(node_feats, WpT, bp_row)


def _msg_body(ef_ref, xs_ref, w1_ref, b1_ref, w2_ref, t_ref, s_ref,
              brow_ref, out_ref):
    dg = lambda a, b: lax.dot_general(a, b, (((1,), (0,)), ((), ())),
                                      preferred_element_type=jnp.float32)
    h = jnp.maximum(ef_ref[...] @ w1_ref[...] + b1_ref[...], 0.0)
    h_bf = h.astype(jnp.bfloat16)
    hh = jnp.concatenate([h_bf, h_bf], axis=1)
    p = dg(hh, w2_ref[...]) + brow_ref[...]
    xs = xs_ref[:, :D]
    xst = dg(xs.astype(jnp.bfloat16), t_ref[...])
    q = xst * p
    acc = dg(q, s_ref[...])
    out_ref[...] = jnp.concatenate(
        [acc, jnp.zeros((acc.shape[0], DP - D), jnp.float32)], axis=1)


def _tc_msg(edge_feats, xs, W1T, be1_row, W2cat, Tt, Sr, Brow):
    return pl.pallas_call(
        _msg_body,
        grid=(EH // EB,),
        in_specs=[
            pl.BlockSpec((EB, D_EDGE), lambda i: (i, 0)),
            pl.BlockSpec((EB, DP), lambda i: (i, 0)),
            pl.BlockSpec((D_EDGE, D_EH), lambda i: (0, 0)),
            pl.BlockSpec((1, D_EH), lambda i: (0, 0)),
            pl.BlockSpec((2 * D_EH, D * D), lambda i: (0, 0)),
            pl.BlockSpec((D, D * D), lambda i: (0, 0)),
            pl.BlockSpec((D * D, D), lambda i: (0, 0)),
            pl.BlockSpec((1, D * D), lambda i: (0, 0)),
        ],
        out_specs=pl.BlockSpec((EB, DP), lambda i: (i, 0)),
        out_shape=jax.ShapeDtypeStruct((EH, DP), jnp.float32),
    )(edge_feats, xs, W1T, be1_row, W2cat, Tt, Sr, Brow)


def _gru_body(*refs):
    agg_refs = refs[:NPART]
    hid_ref, bconv_ref, wih_ref, whh_ref, bih_ref, bhh_ref, out_ref = \
        refs[NPART:]
    hid = hid_ref[:, :D]
    tot = bconv_ref[...]
    for ar in agg_refs:
        tot = tot + ar[0, :, :D] + ar[1, :, :D]
    a = jnp.maximum(tot, 0.0)
    gi = a @ wih_ref[...] + bih_ref[...]
    gh = hid @ whh_ref[...] + bhh_ref[...]
    r = jax.nn.sigmoid(gi[:, :D] + gh[:, :D])
    z = jax.nn.sigmoid(gi[:, D:2 * D] + gh[:, D:2 * D])
    n_ = jnp.tanh(gi[:, 2 * D:] + r * gh[:, 2 * D:])
    x = (1.0 - z) * n_ + z * hid
    out_ref[...] = jnp.concatenate(
        [x, jnp.zeros((x.shape[0], DP - D), jnp.float32)], axis=1)


def _tc_gru(aggs, hid, bconv_row, WihT, WhhT, bih_row, bhh_row):
    in_specs = [pl.BlockSpec((NC, NB, DP), lambda i: (0, i, 0))
                for _ in range(NPART)]
    in_specs += [
        pl.BlockSpec((NB, DP), lambda i: (i, 0)),
        pl.BlockSpec((1, D), lambda i: (0, 0)),
        pl.BlockSpec((D, 3 * D), lambda i: (0, 0)),
        pl.BlockSpec((D, 3 * D), lambda i: (0, 0)),
        pl.BlockSpec((1, 3 * D), lambda i: (0, 0)),
        pl.BlockSpec((1, 3 * D), lambda i: (0, 0)),
    ]
    return pl.pallas_call(
        _gru_body,
        grid=(N // NB,),
        in_specs=in_specs,
        out_specs=pl.BlockSpec((NB, DP), lambda i: (i, 0)),
        out_shape=jax.ShapeDtypeStruct((N, DP), jnp.float32),
    )(*aggs, hid, bconv_row, WihT, WhhT, bih_row, bhh_row)


def _s2s_body(no_ref, w_ih0, w_hh0, b_ih0, b_hh0, w_ih1, w_hh1, b_ih1, b_hh1,
              w_ih2, w_hh2, b_ih2, b_hh2, wo1_ref, bo1_ref, wo2_ref, bo2_ref,
              out_ref):
    no = no_ref[:, :D]
    wih = [w_ih0[...], w_ih1[...], w_ih2[...]]
    whh = [w_hh0[...], w_hh1[...], w_hh2[...]]
    bih = [b_ih0[...], b_ih1[...], b_ih2[...]]
    bhh = [b_hh0[...], b_hh1[...], b_hh2[...]]
    hs = [jnp.zeros((1, D), jnp.float32) for _ in range(3)]
    cs = [jnp.zeros((1, D), jnp.float32) for _ in range(3)]
    q_star = jnp.zeros((1, 2 * D), jnp.float32)
    for _ in range(N_S2S):
        xq = q_star
        for l in range(3):
            gates = xq @ wih[l] + bih[l] + hs[l] @ whh[l] + bhh[l]
            ii = jax.nn.sigmoid(gates[:, :D])
            ff = jax.nn.sigmoid(gates[:, D:2 * D])
            gg = jnp.tanh(gates[:, 2 * D:3 * D])
            oo = jax.nn.sigmoid(gates[:, 3 * D:])
            cs[l] = ff * cs[l] + ii * gg
            hs[l] = oo * jnp.tanh(cs[l])
            xq = hs[l]
        q = xq
        e = lax.dot_general(no, q, (((1,), (1,)), ((), ())))
        al = jnp.exp(e - jnp.max(e))
        al = al / jnp.sum(al)
        readout = lax.dot_general(al, no, (((0,), (0,)), ((), ())))
        q_star = jnp.concatenate([q, readout], axis=1)
    head = jnp.maximum(q_star @ wo1_ref[...] + bo1_ref[...], 0.0)
    out_ref[...] = head @ wo2_ref[...] + bo2_ref[...]


def _tc_s2s(node_out, lstm_t, Wo1T, bo1_row, Wo2T, bo2_row):
    args = [node_out]
    in_specs = [pl.BlockSpec((N, DP), lambda: (0, 0))]
    for (wih, whh, bih, bhh) in lstm_t:
        args += [wih, whh, bih, bhh]
        in_specs += [pl.BlockSpec(wih.shape, lambda: (0, 0)),
                     pl.BlockSpec(whh.shape, lambda: (0, 0)),
                     pl.BlockSpec(bih.shape, lambda: (0, 0)),
                     pl.BlockSpec(bhh.shape, lambda: (0, 0))]
    args += [Wo1T, bo1_row, Wo2T, bo2_row]
    in_specs += [pl.BlockSpec(Wo1T.shape, lambda: (0, 0)),
                 pl.BlockSpec(bo1_row.shape, lambda: (0, 0)),
                 pl.BlockSpec(Wo2T.shape, lambda: (0, 0)),
                 pl.BlockSpec(bo2_row.shape, lambda: (0, 0))]
    return pl.pallas_call(
        _s2s_body,
        grid=(),
        in_specs=in_specs,
        out_specs=pl.BlockSpec((1, 2), lambda: (0, 0)),
        out_shape=jax.ShapeDtypeStruct((1, 2), jnp.float32),
    )(*args)


def kernel(node_feats, edge_feats, edge_index, Wp, bp, We1, be1, We2, be2,
           b_conv, gru_Wih, gru_Whh, gru_bih, gru_bhh,
           lstm_Wih0, lstm_Whh0, lstm_bih0, lstm_bhh0,
           lstm_Wih1, lstm_Whh1, lstm_bih1, lstm_bhh1,
           lstm_Wih2, lstm_Whh2, lstm_bih2, lstm_bhh2,
           Wo1, bo1, Wo2, bo2):
    src = edge_index[0]
    dst = edge_index[1]
    srcP = [src[j * EH:(j + 1) * EH] for j in range(NPART)]
    dstP = [dst[j * EH:(j + 1) * EH] for j in range(NPART)]
    efP = [edge_feats[j * EH:(j + 1) * EH] for j in range(NPART)]
    zeros = jnp.zeros((N, DP), jnp.float32)

    WpT = Wp.T
    W1T = We1.T
    W2m = We2.reshape(D, D, D_EH).transpose(2, 1, 0).reshape(D_EH, D * D)
    W2_hi = W2m.astype(jnp.bfloat16)
    W2_lo = (W2m - W2_hi.astype(jnp.float32)).astype(jnp.bfloat16)
    W2cat = jnp.concatenate([W2_hi, W2_lo], axis=0)
    Tt = jnp.tile(jnp.eye(D, dtype=jnp.bfloat16), (1, D))
    Sr = jnp.repeat(jnp.eye(D, dtype=jnp.float32), D, axis=0)
    Brow = be2.reshape(D, D).T.reshape(1, D * D)
    WihT = gru_Wih.T
    WhhT = gru_Whh.T

    x = _tc_proj(node_feats, WpT, bp.reshape(1, D))
    be1_row = be1.reshape(1, D_EH)
    for _ in range(N_MP):
        xs_cur = _sc_gather(x, srcP[0])
        aggs = []
        for j in range(NPART):
            m = _tc_msg(efP[j], xs_cur, W1T, be1_row, W2cat, Tt, Sr, Brow)
            if j + 1 < NPART:
                xs_cur = _sc_gather(x, srcP[j + 1])
            aggs.append(_sc_scatter(m, dstP[j], zeros))
        x = _tc_gru(aggs, x, b_conv.reshape(1, D), WihT, WhhT,
                    gru_bih.reshape(1, 3 * D), gru_bhh.reshape(1, 3 * D))

    lstm_t = [(lstm_Wih0.T, lstm_Whh0.T, lstm_bih0.reshape(1, 4 * D),
               lstm_bhh0.reshape(1, 4 * D)),
              (lstm_Wih1.T, lstm_Whh1.T, lstm_bih1.reshape(1, 4 * D),
               lstm_bhh1.reshape(1, 4 * D)),
              (lstm_Wih2.T, lstm_Whh2.T, lstm_bih2.reshape(1, 4 * D),
               lstm_bhh2.reshape(1, 4 * D))]
    return _tc_s2s(x, lstm_t, Wo1.T, bo1.reshape(1, D), Wo2.T,
                   bo2.reshape(1, 2))


# speedup vs baseline: 1.3116x; 1.0031x over previous
import jax
import jax.numpy as jnp
from jax import lax
from jax.experimental import pallas as pl
from jax.experimental.pallas import tpu as pltpu
from jax.experimental.pallas import tpu_sc as plsc

N = 10000
E = 160000
D_IN = 128
D_EDGE = 16
D = 32
D_EH = 128
DP = 128
N_MP = 6
N_S2S = 6

NC = 2
NS = 16
NW = NC * NS
NPART = 5
EH = E // NPART
CH = 128
NCHUNK = EH // CH
BASE_CH = NCHUNK // NW
REM = NCHUNK - NW * BASE_CH
OCT = N // 8
BASE_OCT = OCT // NS
REM_OCT = OCT - NS * BASE_OCT
ROWS_SUB = BASE_OCT * 8

EB = 640
NB = 2000

_SC_MESH = plsc.VectorSubcoreMesh(core_axis_name="c", subcore_axis_name="s")


def _gather_body(x_hbm, src_hbm, out_hbm, idx_v, rows_v, sem):
    c = lax.axis_index("c")
    s = lax.axis_index("s")
    w = c * NS + s
    n_ch = BASE_CH + jnp.where(w < REM, 1, 0)
    base = w * BASE_CH + jnp.minimum(w, REM)

    def body(i, carry):
        off = (base + i) * CH
        pltpu.sync_copy(src_hbm.at[pl.ds(off, CH)], idx_v)
        pltpu.async_copy(x_hbm.at[idx_v], rows_v, sem).wait()
        pltpu.sync_copy(rows_v, out_hbm.at[pl.ds(off, CH)])
        return carry

    lax.fori_loop(0, n_ch, body, 0)


_sc_gather_call = pl.kernel(
    _gather_body,
    out_type=jax.ShapeDtypeStruct((EH, DP), jnp.float32),
    mesh=_SC_MESH,
    scratch_types=[
        pltpu.VMEM((CH,), jnp.int32),
        pltpu.VMEM((CH, DP), jnp.float32),
        pltpu.SemaphoreType.DMA,
    ],
)


def _sc_gather(x, src):
    return _sc_gather_call(x, src)


def _scatter_body(m_hbm, dst_hbm, out_hbm, idx_v, rows_v, zb_v, agg_sh):
    c = lax.axis_index("c")
    s = lax.axis_index("s")
    w = c * NS + s
    def zfill(k, carry):
        zb_v[k // 8, pl.ds((k % 8) * 16, 16)] = jnp.zeros((16,), jnp.float32)
        return carry

    lax.fori_loop(0, CH * 8, zfill, 0)
    row0 = (s * BASE_OCT + jnp.minimum(s, REM_OCT)) * 8
    for blk in range(4):
        pltpu.sync_copy(zb_v, agg_sh.at[pl.ds(row0 + blk * CH, CH)])
    pltpu.sync_copy(zb_v.at[pl.ds(0, ROWS_SUB - 4 * CH)],
                    agg_sh.at[pl.ds(row0 + 4 * CH, ROWS_SUB - 4 * CH)])

    @pl.when(s < REM_OCT)
    def _zero_extra():
        pltpu.sync_copy(zb_v.at[pl.ds(0, 8)],
                        agg_sh.at[pl.ds(row0 + ROWS_SUB, 8)])

    plsc.subcore_barrier()

    n_ch = BASE_CH + jnp.where(w < REM, 1, 0)
    base = w * BASE_CH + jnp.minimum(w, REM)

    def body(i, carry):
        off = (base + i) * CH
        pltpu.sync_copy(dst_hbm.at[pl.ds(off, CH)], idx_v)
        pltpu.sync_copy(m_hbm.at[pl.ds(off, CH)], rows_v)
        pltpu.sync_copy(rows_v, agg_sh.at[idx_v], add=True)
        return carry

    lax.fori_loop(0, n_ch, body, 0)
    plsc.subcore_barrier()
    pltpu.sync_copy(agg_sh.at[pl.ds(row0, ROWS_SUB)],
                    out_hbm.at[c, pl.ds(row0, ROWS_SUB)])

    @pl.when(s < REM_OCT)
    def _write_extra():
        pltpu.sync_copy(agg_sh.at[pl.ds(row0 + ROWS_SUB, 8)],
                        out_hbm.at[c, pl.ds(row0 + ROWS_SUB, 8)])


_sc_scatter_call = pl.kernel(
    _scatter_body,
    out_type=jax.ShapeDtypeStruct((NC, N, DP), jnp.float32),
    mesh=_SC_MESH,
    scratch_types=[
        pltpu.VMEM((CH,), jnp.int32),
        pltpu.VMEM((CH, DP), jnp.float32),
        pltpu.VMEM((CH, DP), jnp.float32),
        pltpu.VMEM_SHARED((N, DP), jnp.float32),
    ],
)


def _sc_scatter(m, dst):
    return _sc_scatter_call(m, dst)


def _proj_body(nf_ref, wpt_ref, bp_ref, out_ref):
    x = jnp.maximum(nf_ref[...] @ wpt_ref[...] + bp_ref[...], 0.0)
    out_ref[...] = jnp.concatenate(
        [x, jnp.zeros((x.shape[0], DP - D), jnp.float32)], axis=1)


def _tc_proj(node_feats, WpT, bp_row):
    return pl.pallas_call(
        _proj_body,
        grid=(N // NB,),
        in_specs=[
            pl.BlockSpec((NB, D_IN), lambda i: (i, 0)),
            pl.BlockSpec((D_IN, D), lambda i: (0, 0)),
            pl.BlockSpec((1, D), lambda i: (0, 0)),
        ],
        out_specs=pl.BlockSpec((NB, DP), lambda i: (i, 0)),
        out_shape=jax.ShapeDtypeStruct((N, DP), jnp.float32),
    )(node_feats, WpT, bp_row)


def _msg_body(ef_ref, xs_ref, w1_ref, b1_ref, w2_ref, t_ref, s_ref,
              brow_ref, out_ref):
    dg = lambda a, b: lax.dot_general(a, b, (((1,), (0,)), ((), ())),
                                      preferred_element_type=jnp.float32)
    h = jnp.maximum(ef_ref[...] @ w1_ref[...] + b1_ref[...], 0.0)
    h_bf = h.astype(jnp.bfloat16)
    hh = jnp.concatenate([h_bf, h_bf], axis=1)
    p = dg(hh, w2_ref[...]) + brow_ref[...]
    xs = xs_ref[:, :D]
    xst = dg(xs.astype(jnp.bfloat16), t_ref[...])
    q = xst * p
    acc = dg(q, s_ref[...])
    out_ref[...] = jnp.concatenate(
        [acc, jnp.zeros((acc.shape[0], DP - D), jnp.float32)], axis=1)


def _tc_msg(edge_feats, xs, W1T, be1_row, W2cat, Tt, Sr, Brow):
    return pl.pallas_call(
        _msg_body,
        grid=(EH // EB,),
        in_specs=[
            pl.BlockSpec((EB, D_EDGE), lambda i: (i, 0)),
            pl.BlockSpec((EB, DP), lambda i: (i, 0)),
            pl.BlockSpec((D_EDGE, D_EH), lambda i: (0, 0)),
            pl.BlockSpec((1, D_EH), lambda i: (0, 0)),
            pl.BlockSpec((2 * D_EH, D * D), lambda i: (0, 0)),
            pl.BlockSpec((D, D * D), lambda i: (0, 0)),
            pl.BlockSpec((D * D, D), lambda i: (0, 0)),
            pl.BlockSpec((1, D * D), lambda i: (0, 0)),
        ],
        out_specs=pl.BlockSpec((EB, DP), lambda i: (i, 0)),
        out_shape=jax.ShapeDtypeStruct((EH, DP), jnp.float32),
    )(edge_feats, xs, W1T, be1_row, W2cat, Tt, Sr, Brow)


def _gru_body(*refs):
    agg_refs = refs[:NPART]
    hid_ref, bconv_ref, wih_ref, whh_ref, bih_ref, bhh_ref, out_ref = \
        refs[NPART:]
    hid = hid_ref[:, :D]
    tot = bconv_ref[...]
    for ar in agg_refs:
        tot = tot + ar[0, :, :D] + ar[1, :, :D]
    a = jnp.maximum(tot, 0.0)
    gi = a @ wih_ref[...] + bih_ref[...]
    gh = hid @ whh_ref[...] + bhh_ref[...]
    r = jax.nn.sigmoid(gi[:, :D] + gh[:, :D])
    z = jax.nn.sigmoid(gi[:, D:2 * D] + gh[:, D:2 * D])
    n_ = jnp.tanh(gi[:, 2 * D:] + r * gh[:, 2 * D:])
    x = (1.0 - z) * n_ + z * hid
    out_ref[...] = jnp.concatenate(
        [x, jnp.zeros((x.shape[0], DP - D), jnp.float32)], axis=1)


def _tc_gru(aggs, hid, bconv_row, WihT, WhhT, bih_row, bhh_row):
    in_specs = [pl.BlockSpec((NC, NB, DP), lambda i: (0, i, 0))
                for _ in range(NPART)]
    in_specs += [
        pl.BlockSpec((NB, DP), lambda i: (i, 0)),
        pl.BlockSpec((1, D), lambda i: (0, 0)),
        pl.BlockSpec((D, 3 * D), lambda i: (0, 0)),
        pl.BlockSpec((D, 3 * D), lambda i: (0, 0)),
        pl.BlockSpec((1, 3 * D), lambda i: (0, 0)),
        pl.BlockSpec((1, 3 * D), lambda i: (0, 0)),
    ]
    return pl.pallas_call(
        _gru_body,
        grid=(N // NB,),
        in_specs=in_specs,
        out_specs=pl.BlockSpec((NB, DP), lambda i: (i, 0)),
        out_shape=jax.ShapeDtypeStruct((N, DP), jnp.float32),
    )(*aggs, hid, bconv_row, WihT, WhhT, bih_row, bhh_row)


def _s2s_body(no_ref, w_ih0, w_hh0, b_ih0, b_hh0, w_ih1, w_hh1, b_ih1, b_hh1,
              w_ih2, w_hh2, b_ih2, b_hh2, wo1_ref, bo1_ref, wo2_ref, bo2_ref,
              out_ref):
    no = no_ref[:, :D]
    wih = [w_ih0[...], w_ih1[...], w_ih2[...]]
    whh = [w_hh0[...], w_hh1[...], w_hh2[...]]
    bih = [b_ih0[...], b_ih1[...], b_ih2[...]]
    bhh = [b_hh0[...], b_hh1[...], b_hh2[...]]
    hs = [jnp.zeros((1, D), jnp.float32) for _ in range(3)]
    cs = [jnp.zeros((1, D), jnp.float32) for _ in range(3)]
    q_star = jnp.zeros((1, 2 * D), jnp.float32)
    for _ in range(N_S2S):
        xq = q_star
        for l in range(3):
            gates = xq @ wih[l] + bih[l] + hs[l] @ whh[l] + bhh[l]
            ii = jax.nn.sigmoid(gates[:, :D])
            ff = jax.nn.sigmoid(gates[:, D:2 * D])
            gg = jnp.tanh(gates[:, 2 * D:3 * D])
            oo = jax.nn.sigmoid(gates[:, 3 * D:])
            cs[l] = ff * cs[l] + ii * gg
            hs[l] = oo * jnp.tanh(cs[l])
            xq = hs[l]
        q = xq
        e = lax.dot_general(no, q, (((1,), (1,)), ((), ())))
        al = jnp.exp(e - jnp.max(e))
        al = al / jnp.sum(al)
        readout = lax.dot_general(al, no, (((0,), (0,)), ((), ())))
        q_star = jnp.concatenate([q, readout], axis=1)
    head = jnp.maximum(q_star @ wo1_ref[...] + bo1_ref[...], 0.0)
    out_ref[...] = head @ wo2_ref[...] + bo2_ref[...]


def _tc_s2s(node_out, lstm_t, Wo1T, bo1_row, Wo2T, bo2_row):
    args = [node_out]
    in_specs = [pl.BlockSpec((N, DP), lambda: (0, 0))]
    for (wih, whh, bih, bhh) in lstm_t:
        args += [wih, whh, bih, bhh]
        in_specs += [pl.BlockSpec(wih.shape, lambda: (0, 0)),
                     pl.BlockSpec(whh.shape, lambda: (0, 0)),
                     pl.BlockSpec(bih.shape, lambda: (0, 0)),
                     pl.BlockSpec(bhh.shape, lambda: (0, 0))]
    args += [Wo1T, bo1_row, Wo2T, bo2_row]
    in_specs += [pl.BlockSpec(Wo1T.shape, lambda: (0, 0)),
                 pl.BlockSpec(bo1_row.shape, lambda: (0, 0)),
                 pl.BlockSpec(Wo2T.shape, lambda: (0, 0)),
                 pl.BlockSpec(bo2_row.shape, lambda: (0, 0))]
    return pl.pallas_call(
        _s2s_body,
        grid=(),
        in_specs=in_specs,
        out_specs=pl.BlockSpec((1, 2), lambda: (0, 0)),
        out_shape=jax.ShapeDtypeStruct((1, 2), jnp.float32),
    )(*args)


def kernel(node_feats, edge_feats, edge_index, Wp, bp, We1, be1, We2, be2,
           b_conv, gru_Wih, gru_Whh, gru_bih, gru_bhh,
           lstm_Wih0, lstm_Whh0, lstm_bih0, lstm_bhh0,
           lstm_Wih1, lstm_Whh1, lstm_bih1, lstm_bhh1,
           lstm_Wih2, lstm_Whh2, lstm_bih2, lstm_bhh2,
           Wo1, bo1, Wo2, bo2):
    src = edge_index[0]
    dst = edge_index[1]
    srcP = [src[j * EH:(j + 1) * EH] for j in range(NPART)]
    dstP = [dst[j * EH:(j + 1) * EH] for j in range(NPART)]
    efP = [edge_feats[j * EH:(j + 1) * EH] for j in range(NPART)]

    WpT = Wp.T
    W1T = We1.T
    W2m = We2.reshape(D, D, D_EH).transpose(2, 1, 0).reshape(D_EH, D * D)
    W2_hi = W2m.astype(jnp.bfloat16)
    W2_lo = (W2m - W2_hi.astype(jnp.float32)).astype(jnp.bfloat16)
    W2cat = jnp.concatenate([W2_hi, W2_lo], axis=0)
    Tt = jnp.tile(jnp.eye(D, dtype=jnp.bfloat16), (1, D))
    Sr = jnp.repeat(jnp.eye(D, dtype=jnp.float32), D, axis=0)
    Brow = be2.reshape(D, D).T.reshape(1, D * D)
    WihT = gru_Wih.T
    WhhT = gru_Whh.T

    x = _tc_proj(node_feats, WpT, bp.reshape(1, D))
    be1_row = be1.reshape(1, D_EH)
    for _ in range(N_MP):
        xs_cur = _sc_gather(x, srcP[0])
        aggs = []
        for j in range(NPART):
            m = _tc_msg(efP[j], xs_cur, W1T, be1_row, W2cat, Tt, Sr, Brow)
            if j + 1 < NPART:
                xs_cur = _sc_gather(x, srcP[j + 1])
            aggs.append(_sc_scatter(m, dstP[j]))
        x = _tc_gru(aggs, x, b_conv.reshape(1, D), WihT, WhhT,
                    gru_bih.reshape(1, 3 * D), gru_bhh.reshape(1, 3 * D))

    lstm_t = [(lstm_Wih0.T, lstm_Whh0.T, lstm_bih0.reshape(1, 4 * D),
               lstm_bhh0.reshape(1, 4 * D)),
              (lstm_Wih1.T, lstm_Whh1.T, lstm_bih1.reshape(1, 4 * D),
               lstm_bhh1.reshape(1, 4 * D)),
              (lstm_Wih2.T, lstm_Whh2.T, lstm_bih2.reshape(1, 4 * D),
               lstm_bhh2.reshape(1, 4 * D))]
    return _tc_s2s(x, lstm_t, Wo1.T, bo1.reshape(1, D), Wo2.T,
                   bo2.reshape(1, 2))
